# baseline (speedup 1.0000x reference)
import jax
import jax.numpy as jnp
from jax import lax
from jax.experimental import pallas as pl
from jax.experimental.pallas import tpu as pltpu
from jax.experimental.pallas import tpu_sc as plsc

_B = 16384
_D = 64
_V = 1000000
_NC = 2
_NS = 16
_NW = _NC * _NS
_BW = _B // _NW
_CH = 128
_NCH = _BW // _CH
_L = 16


def _mf_body(uid, iid, uflat, iflat, ubias, ibias, gbias, out,
             uidx, iidx, uR, iR, ub, ib, gb, outv, pacc, sem):
    c = lax.axis_index("c")
    s = lax.axis_index("s")
    base = (s * _NC + c) * _BW
    iota = lax.iota(jnp.int32, _L)

    pltpu.sync_copy(uid.at[pl.ds(base, _BW)], uidx)
    pltpu.sync_copy(iid.at[pl.ds(base, _BW)], iidx)
    pltpu.sync_copy(gbias, gb)

    for k in range(_NCH):
        sl = pl.ds(k * _CH, _CH)
        pltpu.async_copy(ubias.at[uidx.at[sl]], ub.at[sl], sem)
        pltpu.async_copy(ibias.at[iidx.at[sl]], ib.at[sl], sem)

    def fire(g, carry):
        uv = uidx[pl.ds(g * _L, _L)]
        iv = iidx[pl.ds(g * _L, _L)]
        for lane in range(_L):
            dsl = pl.ds((g * _L + lane) * _D, _D)
            for r, flat, dst in ((uv[lane], uflat, uR), (iv[lane], iflat, iR)):
                hi = (r >= _HALF).astype(jnp.int32)
                row = r - hi * _HALF
                col = hi * _D
                pltpu.async_copy(flat.at[row, pl.ds(col, _D)], dst.at[dsl], sem)
        return carry

    lax.fori_loop(0, _BW // _L, fire, 0)

    pltpu.make_async_copy(ubias.at[pl.ds(0, _BW * _D)], uR, sem).wait()
    pltpu.make_async_copy(ubias.at[pl.ds(0, _BW * _D)], iR, sem).wait()
    pltpu.make_async_copy(ubias.at[pl.ds(0, _BW)], ub, sem).wait()
    pltpu.make_async_copy(ibias.at[pl.ds(0, _BW)], ib, sem).wait()

    gvec = gb[...]
    rowsel = iota * _L

    def group(g, carry):
        osl = pl.ds(g * _L, _L)
        for r in range(_L):
            rbase = (g * _L + r) * _D
            acc = None
            for cc in range(_D // _L):
                csl = pl.ds(rbase + cc * _L, _L)
                p = uR[csl] * iR[csl]
                acc = p if acc is None else acc + p
            pacc[pl.ds(r * _L, _L)] = acc
        tot = (ub[osl] + ib[osl]) + gvec
        for cc in range(_L):
            tot = tot + plsc.load_gather(pacc, [rowsel + cc])
        outv[osl] = tot
        return carry

    lax.fori_loop(0, _BW // _L, group, 0)
    pltpu.sync_copy(outv, out.at[pl.ds(base, _BW)])


_TBLK = 512
_TGRID = (_V + 2 * _TBLK - 1) // (2 * _TBLK)
_HALF = _TGRID * _TBLK


def _tr_body(xa_ref, xb_ref, o_ref):
    eye = (lax.broadcasted_iota(jnp.int32, (_D, _D), 0)
           == lax.broadcasted_iota(jnp.int32, (_D, _D), 1)).astype(jnp.float32)
    dn = (((0,), (0,)), ((), ()))
    xa = xa_ref[...]
    xb = xb_ref[...]
    ya = lax.dot_general(xa, eye, dn, preferred_element_type=jnp.float32)
    yb = lax.dot_general(xb, eye, dn, preferred_element_type=jnp.float32)
    o_ref[...] = jnp.concatenate([ya, yb], axis=1)


def _flatten(embT):
    return pl.pallas_call(
        _tr_body,
        grid=(_TGRID,),
        in_specs=[
            pl.BlockSpec((_D, _TBLK), lambda g: (0, g)),
            pl.BlockSpec((_D, _TBLK), lambda g: (0, g + _TGRID)),
        ],
        out_specs=pl.BlockSpec((_TBLK, 2 * _D), lambda g: (g, 0)),
        out_shape=jax.ShapeDtypeStruct((_HALF, 2 * _D), jnp.float32),
    )(embT, embT)


def kernel(user_ids, item_ids, user_emb, item_emb, user_bias, item_bias, global_bias):
    uid = user_ids.astype(jnp.int32)
    iid = item_ids.astype(jnp.int32)
    uflat = _flatten(user_emb.T)
    iflat = _flatten(item_emb.T)
    ubias = user_bias.reshape(-1)
    ibias = item_bias.reshape(-1)
    gb16 = jnp.broadcast_to(global_bias.astype(jnp.float32), (_L,))
    mesh = plsc.VectorSubcoreMesh(core_axis_name="c", subcore_axis_name="s")
    f = pl.kernel(
        _mf_body,
        mesh=mesh,
        compiler_params=pltpu.CompilerParams(
            needs_layout_passes=False, use_tc_tiling_on_sc=False),
        out_type=jax.ShapeDtypeStruct((_B,), jnp.float32),
        scratch_types=[
            pltpu.VMEM((_BW,), jnp.int32),
            pltpu.VMEM((_BW,), jnp.int32),
            pltpu.VMEM((_BW * _D,), jnp.float32),
            pltpu.VMEM((_BW * _D,), jnp.float32),
            pltpu.VMEM((_BW,), jnp.float32),
            pltpu.VMEM((_BW,), jnp.float32),
            pltpu.VMEM((_L,), jnp.float32),
            pltpu.VMEM((_BW,), jnp.float32),
            pltpu.VMEM((_L * _L,), jnp.float32),
            pltpu.SemaphoreType.DMA,
        ],
    )
    return f(uid, iid, uflat, iflat, ubias, ibias, gb16)


# speedup vs baseline: 1.7948x; 1.7948x over previous
import jax
import jax.numpy as jnp
from jax import lax
from jax.experimental import pallas as pl
from jax.experimental.pallas import tpu as pltpu
from jax.experimental.pallas import tpu_sc as plsc

_B = 16384
_D = 64
_V = 1000000
_NC = 2
_NS = 16
_NW = _NC * _NS
_BW = _B // _NW
_CH = 128
_NCH = _BW // _CH
_L = 16
_RING = 16


def _mf_body(uid, iid, uemb, iemb, ubias, ibias, gbias, out,
             uidx, iidx, uring, iring, uR, iR, ub, ib, gb, outv, pacc,
             semu, semi, semb):
    c = lax.axis_index("c")
    s = lax.axis_index("s")
    base = (s * _NC + c) * _BW
    iota = lax.iota(jnp.int32, _L)

    pltpu.sync_copy(uid.at[pl.ds(base, _BW)], uidx)
    pltpu.sync_copy(iid.at[pl.ds(base, _BW)], iidx)
    pltpu.sync_copy(gbias, gb)

    for k in range(_NCH):
        sl = pl.ds(k * _CH, _CH)
        pltpu.async_copy(ubias.at[uidx.at[sl]], ub.at[sl], semb)
        pltpu.async_copy(ibias.at[iidx.at[sl]], ib.at[sl], semb)

    cols = [cc * _L + iota for cc in range(_D // _L)]

    def extract(j, idxr, ring, dst, slot):
        r = plsc.load_gather(idxr, [j + iota * 0])
        rlo = r & 7
        for cc in range(_D // _L):
            v = plsc.load_gather(ring, [slot + rlo * 0, rlo, cols[cc]])
            plsc.store_scatter(dst, [j * _D + cols[cc]], v)

    def drain_and_extract(g):
        for slot in range(_RING):
            pltpu.make_async_copy(uemb.at[pl.ds(0, 8), :], uring.at[slot], semu).wait()
            pltpu.make_async_copy(iemb.at[pl.ds(0, 8), :], iring.at[slot], semi).wait()
        for slot in range(_RING):
            j = g * _L + slot
            extract(j, uidx, uring, uR, slot)
            extract(j, iidx, iring, iR, slot)

    def fire(g, carry):
        uv = uidx[pl.ds(g * _L, _L)]
        iv = iidx[pl.ds(g * _L, _L)]

        @pl.when(g > 0)
        def _():
            drain_and_extract(g - 1)

        for lane in range(_L):
            ru = pl.multiple_of((uv[lane] >> 3) * 8, 8)
            ri = pl.multiple_of((iv[lane] >> 3) * 8, 8)
            pltpu.async_copy(uemb.at[pl.ds(ru, 8), :], uring.at[lane], semu)
            pltpu.async_copy(iemb.at[pl.ds(ri, 8), :], iring.at[lane], semi)
        return carry

    lax.fori_loop(0, _BW // _L, fire, 0)
    drain_and_extract(_BW // _L - 1)

    pltpu.make_async_copy(ubias.at[pl.ds(0, _BW)], ub, semb).wait()
    pltpu.make_async_copy(ibias.at[pl.ds(0, _BW)], ib, semb).wait()

    gvec = gb[...]
    rowsel = iota * _L

    def group(g, carry):
        osl = pl.ds(g * _L, _L)
        for r in range(_L):
            rbase = (g * _L + r) * _D
            acc = None
            for cc in range(_D // _L):
                csl = pl.ds(rbase + cc * _L, _L)
                p = uR[csl] * iR[csl]
                acc = p if acc is None else acc + p
            pacc[pl.ds(r * _L, _L)] = acc
        tot = (ub[osl] + ib[osl]) + gvec
        for cc in range(_L):
            tot = tot + plsc.load_gather(pacc, [rowsel + cc])
        outv[osl] = tot
        return carry

    lax.fori_loop(0, _BW // _L, group, 0)
    pltpu.sync_copy(outv, out.at[pl.ds(base, _BW)])


def kernel(user_ids, item_ids, user_emb, item_emb, user_bias, item_bias, global_bias):
    uid = user_ids.astype(jnp.int32)
    iid = item_ids.astype(jnp.int32)
    ubias = user_bias.reshape(-1)
    ibias = item_bias.reshape(-1)
    gb16 = jnp.broadcast_to(global_bias.astype(jnp.float32), (_L,))
    mesh = plsc.VectorSubcoreMesh(core_axis_name="c", subcore_axis_name="s")
    f = pl.kernel(
        _mf_body,
        mesh=mesh,
        compiler_params=pltpu.CompilerParams(needs_layout_passes=False),
        out_type=jax.ShapeDtypeStruct((_B,), jnp.float32),
        scratch_types=[
            pltpu.VMEM((_BW,), jnp.int32),
            pltpu.VMEM((_BW,), jnp.int32),
            pltpu.VMEM((_RING, 8, _D), jnp.float32),
            pltpu.VMEM((_RING, 8, _D), jnp.float32),
            pltpu.VMEM((_BW * _D,), jnp.float32),
            pltpu.VMEM((_BW * _D,), jnp.float32),
            pltpu.VMEM((_BW,), jnp.float32),
            pltpu.VMEM((_BW,), jnp.float32),
            pltpu.VMEM((_L,), jnp.float32),
            pltpu.VMEM((_BW,), jnp.float32),
            pltpu.VMEM((_L * _L,), jnp.float32),
            pltpu.SemaphoreType.DMA,
            pltpu.SemaphoreType.DMA,
            pltpu.SemaphoreType.DMA,
        ],
    )
    return f(uid, iid, user_emb, item_emb, ubias, ibias, gb16)


# speedup vs baseline: 1.8497x; 1.0306x over previous
import jax
import jax.numpy as jnp
from jax import lax
from jax.experimental import pallas as pl
from jax.experimental.pallas import tpu as pltpu
from jax.experimental.pallas import tpu_sc as plsc

_B = 16384
_D = 64
_NC = 2
_NS = 16
_NW = _NC * _NS
_BW = _B // _NW
_CH = 128
_NCH = _BW // _CH
_L = 16
_NG = _BW // _L


def _mf_body(uid, iid, uemb, iemb, ubias, ibias, gbias, out,
             uidx, iidx, uring, iring, uRg, iRg, ub, ib, gb, outv, pacc,
             semu, semi, semb):
    c = lax.axis_index("c")
    s = lax.axis_index("s")
    base = (s * _NC + c) * _BW
    iota = lax.iota(jnp.int32, _L)

    pltpu.sync_copy(uid.at[pl.ds(base, _BW)], uidx)
    pltpu.sync_copy(iid.at[pl.ds(base, _BW)], iidx)
    pltpu.sync_copy(gbias, gb)

    for k in range(_NCH):
        sl = pl.ds(k * _CH, _CH)
        pltpu.async_copy(ubias.at[uidx.at[sl]], ub.at[sl], semb)
        pltpu.async_copy(ibias.at[iidx.at[sl]], ib.at[sl], semb)
    pltpu.make_async_copy(ubias.at[pl.ds(0, _BW)], ub, semb).wait()
    pltpu.make_async_copy(ibias.at[pl.ds(0, _BW)], ib, semb).wait()

    gvec = gb[...]
    rowsel = iota * _L
    cols = [cc * _L + iota for cc in range(_D // _L)]

    def extract(j, slot, idxr, ring, dst, lane):
        r = plsc.load_gather(idxr, [j + iota * 0])
        rlo = r & 7
        for cc in range(_D // _L):
            v = plsc.load_gather(ring, [slot + rlo * 0, rlo, cols[cc]])
            plsc.store_scatter(dst, [lane * _D + cols[cc]], v)

    def consume(g, half):
        for _ in range(_L):
            pltpu.make_async_copy(uemb.at[pl.ds(0, 8), :], uring.at[0], semu).wait()
            pltpu.make_async_copy(iemb.at[pl.ds(0, 8), :], iring.at[0], semi).wait()
        for lane in range(_L):
            j = g * _L + lane
            extract(j, half * _L + lane, uidx, uring, uRg, lane)
            extract(j, half * _L + lane, iidx, iring, iRg, lane)
        for lane in range(_L):
            acc = None
            for cc in range(_D // _L):
                csl = pl.ds(lane * _D + cc * _L, _L)
                p = uRg[csl] * iRg[csl]
                acc = p if acc is None else acc + p
            pacc[pl.ds(lane * _L, _L)] = acc
        osl = pl.ds(g * _L, _L)
        tot = (ub[osl] + ib[osl]) + gvec
        for cc in range(_L):
            tot = tot + plsc.load_gather(pacc, [rowsel + cc])
        outv[osl] = tot

    def fire(g, carry):
        uv = uidx[pl.ds(g * _L, _L)]
        iv = iidx[pl.ds(g * _L, _L)]
        half = g & 1
        for lane in range(_L):
            ru = pl.multiple_of((uv[lane] >> 3) * 8, 8)
            ri = pl.multiple_of((iv[lane] >> 3) * 8, 8)
            pltpu.async_copy(uemb.at[pl.ds(ru, 8), :], uring.at[half * _L + lane], semu)
            pltpu.async_copy(iemb.at[pl.ds(ri, 8), :], iring.at[half * _L + lane], semi)

        @pl.when(g > 0)
        def _():
            consume(g - 1, 1 - half)

        return carry

    lax.fori_loop(0, _NG, fire, 0)
    consume(_NG - 1, (_NG - 1) & 1)

    pltpu.sync_copy(outv, out.at[pl.ds(base, _BW)])


def kernel(user_ids, item_ids, user_emb, item_emb, user_bias, item_bias, global_bias):
    uid = user_ids.astype(jnp.int32)
    iid = item_ids.astype(jnp.int32)
    ubias = user_bias.reshape(-1)
    ibias = item_bias.reshape(-1)
    gb16 = jnp.broadcast_to(global_bias.astype(jnp.float32), (_L,))
    mesh = plsc.VectorSubcoreMesh(core_axis_name="c", subcore_axis_name="s")
    f = pl.kernel(
        _mf_body,
        mesh=mesh,
        compiler_params=pltpu.CompilerParams(needs_layout_passes=False),
        out_type=jax.ShapeDtypeStruct((_B,), jnp.float32),
        scratch_types=[
            pltpu.VMEM((_BW,), jnp.int32),
            pltpu.VMEM((_BW,), jnp.int32),
            pltpu.VMEM((2 * _L, 8, _D), jnp.float32),
            pltpu.VMEM((2 * _L, 8, _D), jnp.float32),
            pltpu.VMEM((_L * _D,), jnp.float32),
            pltpu.VMEM((_L * _D,), jnp.float32),
            pltpu.VMEM((_BW,), jnp.float32),
            pltpu.VMEM((_BW,), jnp.float32),
            pltpu.VMEM((_L,), jnp.float32),
            pltpu.VMEM((_BW,), jnp.float32),
            pltpu.VMEM((_L * _L,), jnp.float32),
            pltpu.SemaphoreType.DMA,
            pltpu.SemaphoreType.DMA,
            pltpu.SemaphoreType.DMA,
        ],
    )
    return f(uid, iid, user_emb, item_emb, ubias, ibias, gb16)


# speedup vs baseline: 1.8745x; 1.0134x over previous
import jax
import jax.numpy as jnp
from jax import lax
from jax.experimental import pallas as pl
from jax.experimental.pallas import tpu as pltpu
from jax.experimental.pallas import tpu_sc as plsc

_B = 16384
_V = 1000000
_D = 64
_NC = 2
_NS = 16
_NW = _NC * _NS
_BW = _B // _NW
_CH = 128
_NCH = _BW // _CH
_L = 16
_NG = _BW // _L


def _mf_body(uid, iid, uemb, iemb, ubias, ibias, gbias, out,
             uidx, iidx, uring, iring, uRg, iRg, ub, ib, gb, outv,
             pacc, semu, semi, semb):
    c = lax.axis_index("c")
    s = lax.axis_index("s")
    base = (s * _NC + c) * _BW
    iota = lax.iota(jnp.int32, _L)

    pltpu.sync_copy(uid.at[pl.ds(base, _BW)], uidx)
    pltpu.sync_copy(iid.at[pl.ds(base, _BW)], iidx)
    pltpu.sync_copy(gbias, gb)

    for k in range(_NCH):
        sl = pl.ds(k * _CH, _CH)
        pltpu.async_copy(ubias.at[uidx.at[sl]], ub.at[sl], semb)
        pltpu.async_copy(ibias.at[iidx.at[sl]], ib.at[sl], semb)
    pltpu.make_async_copy(ubias.at[pl.ds(0, _BW)], ub, semb).wait()
    pltpu.make_async_copy(ibias.at[pl.ds(0, _BW)], ib, semb).wait()


    gvec = gb[...]
    rowsel = iota * _L
    cols = [cc * _L + iota for cc in range(_D // _L)]

    def extract(j, slot, idxr, ring, dst, lane):
        r = plsc.load_gather(idxr, [j + iota * 0])
        rlo = r & 7
        for cc in range(_D // _L):
            v = plsc.load_gather(ring, [slot + rlo * 0, rlo, cols[cc]])
            plsc.store_scatter(dst, [lane * _D + cols[cc]], v)

    def consume(g, half):
        for _ in range(_L):
            pltpu.make_async_copy(uemb.at[pl.ds(0, 8), :], uring.at[0], semu).wait()
            pltpu.make_async_copy(iemb.at[pl.ds(0, 8), :], iring.at[0], semi).wait()
        for lane in range(_L):
            j = g * _L + lane
            extract(j, half * _L + lane, uidx, uring, uRg, lane)
            extract(j, half * _L + lane, iidx, iring, iRg, lane)
        for lane in range(_L):
            acc = None
            for cc in range(_D // _L):
                csl = pl.ds(lane * _D + cc * _L, _L)
                p = uRg[csl] * iRg[csl]
                acc = p if acc is None else acc + p
            pacc[pl.ds(lane * _L, _L)] = acc
        osl = pl.ds(g * _L, _L)
        tot = (ub[osl] + ib[osl]) + gvec
        for cc in range(_L):
            tot = tot + plsc.load_gather(pacc, [rowsel + cc])
        outv[osl] = tot

    def fire(g, carry):
        uv = uidx[pl.ds(g * _L, _L)]
        iv = iidx[pl.ds(g * _L, _L)]
        slot0 = lax.rem(g, 3) * _L
        for lane in range(_L):
            ru = pl.multiple_of((uv[lane] >> 3) * 8, 8)
            ri = pl.multiple_of((iv[lane] >> 3) * 8, 8)
            pltpu.async_copy(uemb.at[pl.ds(ru, 8), :], uring.at[slot0 + lane], semu)
            pltpu.async_copy(iemb.at[pl.ds(ri, 8), :], iring.at[slot0 + lane], semi)

        @pl.when(g > 1)
        def _():
            consume(g - 2, lax.rem(g - 2, 3))

        return carry

    lax.fori_loop(0, _NG, fire, 0)
    consume(_NG - 2, (_NG - 2) % 3)
    consume(_NG - 1, (_NG - 1) % 3)

    pltpu.sync_copy(outv, out.at[pl.ds(base, _BW)])


def kernel(user_ids, item_ids, user_emb, item_emb, user_bias, item_bias, global_bias):
    uid = user_ids.astype(jnp.int32)
    iid = item_ids.astype(jnp.int32)
    ubias = user_bias.reshape(-1)
    ibias = item_bias.reshape(-1)
    gb16 = jnp.broadcast_to(global_bias.astype(jnp.float32), (_L,))
    mesh = plsc.VectorSubcoreMesh(core_axis_name="c", subcore_axis_name="s")
    f = pl.kernel(
        _mf_body,
        mesh=mesh,
        compiler_params=pltpu.CompilerParams(needs_layout_passes=False),
        out_type=jax.ShapeDtypeStruct((_B,), jnp.float32),
        scratch_types=[
            pltpu.VMEM((_BW,), jnp.int32),
            pltpu.VMEM((_BW,), jnp.int32),
            pltpu.VMEM((3 * _L, 8, _D), jnp.float32),
            pltpu.VMEM((3 * _L, 8, _D), jnp.float32),
            pltpu.VMEM((_L * _D,), jnp.float32),
            pltpu.VMEM((_L * _D,), jnp.float32),
            pltpu.VMEM((_BW,), jnp.float32),
            pltpu.VMEM((_BW,), jnp.float32),
            pltpu.VMEM((_L,), jnp.float32),
            pltpu.VMEM((_BW,), jnp.float32),
            pltpu.VMEM((_L * _L,), jnp.float32),
            pltpu.SemaphoreType.DMA,
            pltpu.SemaphoreType.DMA,
            pltpu.SemaphoreType.DMA,
        ],
    )
    return f(uid, iid, user_emb, item_emb, ubias, ibias, gb16)
